# baseline (speedup 1.0000x reference)
import jax
import jax.numpy as jnp
from jax import lax
from jax.experimental import pallas as pl
from jax.experimental.pallas import tpu as pltpu
from jax.experimental.pallas import tpu_sc as plsc

CHARGE_DIM = 128
BATCH = 16
TOTAL_NODES = 32768

NUM_CORES = 2
NUM_SUBCORES = 16
LANES = 16
NUM_WORKERS = NUM_CORES * NUM_SUBCORES
ROWS_PER_WORKER = TOTAL_NODES // NUM_WORKERS
CHUNK_ROWS = 128
NUM_CHUNKS = ROWS_PER_WORKER // CHUNK_ROWS
COLS = CHARGE_DIM // LANES


def _sc_body(charge_hbm, seg_hbm, out_hbm, charge_v, seg_v, buf0, buf1,
             sem0, sem1):
    wid = lax.axis_index("s") * NUM_CORES + lax.axis_index("c")
    base = wid * ROWS_PER_WORKER

    pltpu.sync_copy(charge_hbm, charge_v)
    pltpu.sync_copy(seg_hbm.at[pl.ds(base, ROWS_PER_WORKER)],
                    seg_v.at[pl.ds(0, ROWS_PER_WORKER)])

    charge_reg = charge_v[...]
    zeros16 = jnp.zeros((LANES,), jnp.int32)

    def fast_fill(buf, cval):
        row = jnp.full((LANES,), cval, dtype=jnp.float32)

        def body(i, _):
            for j in range(COLS):
                buf[pl.ds(i * CHARGE_DIM + j * LANES, LANES)] = row
            return ()

        lax.fori_loop(0, CHUNK_ROWS, body, (), unroll=2)

    def perrow_fill(buf, row0):
        def body(i, _):
            sv = seg_v[pl.ds(row0 + i, LANES)]
            sid = sv.at[zeros16].get(mode="promise_in_bounds")
            row = charge_reg.at[sid].get(mode="promise_in_bounds")
            for j in range(COLS):
                buf[pl.ds(i * CHARGE_DIM + j * LANES, LANES)] = row
            return ()

        lax.fori_loop(0, CHUNK_ROWS, body, (), unroll=1)

    bufs = (buf0, buf1)
    sems = (sem0, sem1)
    copies = [None, None]
    valid = [jnp.bool_(False), jnp.bool_(False)]
    cur_val = [jnp.float32(0.0), jnp.float32(0.0)]
    for chunk in range(NUM_CHUNKS):
        b = chunk % 2
        row0 = chunk * CHUNK_ROWS
        sv0 = seg_v[pl.ds(row0, LANES)]
        svl = seg_v[pl.ds(row0 + CHUNK_ROWS - LANES, LANES)]
        uniform = sv0[0] == svl[LANES - 1]
        cval = charge_reg.at[sv0].get(mode="promise_in_bounds")[0]
        skip = uniform & valid[b] & (cval == cur_val[b])
        if copies[b] is not None:
            copies[b].wait()

        @pl.when(jnp.logical_not(skip) & uniform)
        def _():
            fast_fill(bufs[b], cval)

        @pl.when(jnp.logical_not(uniform))
        def _():
            perrow_fill(bufs[b], row0)

        valid[b] = uniform
        cur_val[b] = cval
        copies[b] = pltpu.async_copy(
            bufs[b],
            out_hbm.at[pl.ds((base + row0) * CHARGE_DIM,
                             CHUNK_ROWS * CHARGE_DIM)],
            sems[b])
    for b in range(2):
        copies[b].wait()


_sc_kernel = pl.kernel(
    _sc_body,
    out_type=jax.ShapeDtypeStruct((TOTAL_NODES * CHARGE_DIM,), jnp.float32),
    mesh=plsc.VectorSubcoreMesh(core_axis_name="c", subcore_axis_name="s"),
    scratch_types=[
        pltpu.VMEM((BATCH,), jnp.float32),
        pltpu.VMEM((ROWS_PER_WORKER + LANES,), jnp.int32),
        pltpu.VMEM((CHUNK_ROWS * CHARGE_DIM,), jnp.float32),
        pltpu.VMEM((CHUNK_ROWS * CHARGE_DIM,), jnp.float32),
        pltpu.SemaphoreType.DMA,
        pltpu.SemaphoreType.DMA,
    ],
)


def kernel(charge, segment_ids):
    seg = segment_ids.astype(jnp.int32)
    out = _sc_kernel(charge.astype(jnp.float32), seg)
    return out.reshape(TOTAL_NODES, CHARGE_DIM)


# speedup vs baseline: 1.0124x; 1.0124x over previous
import jax
import jax.numpy as jnp
from jax import lax
from jax.experimental import pallas as pl
from jax.experimental.pallas import tpu as pltpu
from jax.experimental.pallas import tpu_sc as plsc

CHARGE_DIM = 128
BATCH = 16
TOTAL_NODES = 32768

NUM_CORES = 2
NUM_SUBCORES = 16
LANES = 16
NUM_WORKERS = NUM_CORES * NUM_SUBCORES
ROWS_PER_WORKER = TOTAL_NODES // NUM_WORKERS
BUF_ROWS = 128
COLS = CHARGE_DIM // LANES

CHUNK_SCHED = [(0, 32), (32, 128), (160, 128), (288, 128), (416, 128),
               (544, 128), (672, 128), (800, 128), (928, 96)]
assert CHUNK_SCHED[-1][0] + CHUNK_SCHED[-1][1] == ROWS_PER_WORKER


def _sc_body(charge_hbm, seg_hbm, out_hbm, charge_v, seg_v, buf0, buf1,
             semc, sems_, sem0, sem1):
    wid = lax.axis_index("s") * NUM_CORES + lax.axis_index("c")
    base = wid * ROWS_PER_WORKER

    in0 = pltpu.async_copy(charge_hbm, charge_v, semc)
    in1 = pltpu.async_copy(seg_hbm.at[pl.ds(base, ROWS_PER_WORKER)],
                           seg_v.at[pl.ds(0, ROWS_PER_WORKER)], sems_)
    in0.wait()
    in1.wait()

    charge_reg = charge_v[...]
    zeros16 = jnp.zeros((LANES,), jnp.int32)

    def fast_fill(buf, cval, start, rows):
        row = jnp.full((LANES,), cval, dtype=jnp.float32)

        def body(i, _):
            for j in range(COLS):
                buf[pl.ds(i * CHARGE_DIM + j * LANES, LANES)] = row
            return ()

        lax.fori_loop(start, rows, body, ())

    def perrow_fill(buf, row0, rows):
        def body(i, _):
            sv = seg_v[pl.ds(row0 + i, LANES)]
            sid = sv.at[zeros16].get(mode="promise_in_bounds")
            row = charge_reg.at[sid].get(mode="promise_in_bounds")
            for j in range(COLS):
                buf[pl.ds(i * CHARGE_DIM + j * LANES, LANES)] = row
            return ()

        lax.fori_loop(0, rows, body, ())

    bufs = (buf0, buf1)
    sems = (sem0, sem1)
    copies = [None, None]
    valid = [jnp.bool_(False), jnp.bool_(False)]
    cur_val = [jnp.float32(0.0), jnp.float32(0.0)]
    filled = [jnp.int32(0), jnp.int32(0)]
    for chunk, (row0, rows) in enumerate(CHUNK_SCHED):
        b = chunk % 2
        sv0 = seg_v[pl.ds(row0, LANES)]
        svl = seg_v[pl.ds(row0 + rows - LANES, LANES)]
        uniform = sv0[0] == svl[LANES - 1]
        cval = charge_reg.at[sv0].get(mode="promise_in_bounds")[0]
        same = valid[b] & (cval == cur_val[b])
        start = jnp.where(same, jnp.minimum(filled[b], rows), 0)
        if copies[b] is not None:
            copies[b].wait()

        @pl.when(uniform & (start < rows))
        def _():
            fast_fill(bufs[b], cval, start, rows)

        @pl.when(jnp.logical_not(uniform))
        def _():
            perrow_fill(bufs[b], row0, rows)

        filled[b] = jnp.where(
            uniform, jnp.where(same, jnp.maximum(filled[b], rows), rows), 0)
        valid[b] = uniform
        cur_val[b] = cval
        copies[b] = pltpu.async_copy(
            bufs[b].at[pl.ds(0, rows * CHARGE_DIM)],
            out_hbm.at[pl.ds((base + row0) * CHARGE_DIM, rows * CHARGE_DIM)],
            sems[b])
    for b in range(2):
        copies[b].wait()


_sc_kernel = pl.kernel(
    _sc_body,
    out_type=jax.ShapeDtypeStruct((TOTAL_NODES * CHARGE_DIM,), jnp.float32),
    mesh=plsc.VectorSubcoreMesh(core_axis_name="c", subcore_axis_name="s"),
    scratch_types=[
        pltpu.VMEM((BATCH,), jnp.float32),
        pltpu.VMEM((ROWS_PER_WORKER + LANES,), jnp.int32),
        pltpu.VMEM((BUF_ROWS * CHARGE_DIM,), jnp.float32),
        pltpu.VMEM((BUF_ROWS * CHARGE_DIM,), jnp.float32),
        pltpu.SemaphoreType.DMA,
        pltpu.SemaphoreType.DMA,
        pltpu.SemaphoreType.DMA,
        pltpu.SemaphoreType.DMA,
    ],
)


def kernel(charge, segment_ids):
    seg = segment_ids.astype(jnp.int32)
    out = _sc_kernel(charge.astype(jnp.float32), seg)
    return out.reshape(TOTAL_NODES, CHARGE_DIM)
